# baseline (speedup 1.0000x reference)
import jax
import jax.numpy as jnp
from jax.experimental import pallas as pl
from jax.experimental.pallas import tpu as pltpu


_RB = 128
_NSLOT = 3
_VB = 256
_G1 = 512
_G2 = 1024


def _kern(
    v_hbm,
    adj_hbm,
    w1_ref,
    w2_ref,
    wout_ref,
    bout_ref,
    out_ref,
    adjb_ref,
    z1_ref,
    z2_ref,
    vbuf_ref,
    abuf_ref,
    w1c_ref,
    w2c_ref,
    vsem,
    asem,
):
    N = adj_hbm.shape[0]
    nv = N // _VB
    nb = N // _RB
    ng1 = N // _G1
    ng2 = N // _G2

    def _start_adj(t):
        pltpu.make_async_copy(
            adj_hbm.at[pl.ds(t * _RB, _RB), :],
            abuf_ref.at[t % _NSLOT],
            asem.at[t % _NSLOT],
        ).start()

    def _wait_adj(t):
        pltpu.make_async_copy(
            adj_hbm.at[pl.ds(t * _RB, _RB), :],
            abuf_ref.at[t % _NSLOT],
            asem.at[t % _NSLOT],
        ).wait()

    _start_adj(0)
    _start_adj(1)
    pltpu.make_async_copy(
        v_hbm.at[pl.ds(0, _VB), :], vbuf_ref.at[0], vsem.at[0]
    ).start()
    w1c_ref[...] = w1_ref[...].astype(jnp.bfloat16)
    w2c_ref[...] = w2_ref[...].astype(jnp.bfloat16)

    for t in range(nv):
        if t + 1 < nv:
            pltpu.make_async_copy(
                v_hbm.at[pl.ds((t + 1) * _VB, _VB), :],
                vbuf_ref.at[(t + 1) % 2],
                vsem.at[(t + 1) % 2],
            ).start()
        pltpu.make_async_copy(
            v_hbm.at[pl.ds(t * _VB, _VB), :], vbuf_ref.at[t % 2], vsem.at[t % 2]
        ).wait()
        z1_ref[pl.ds(t * _VB, _VB), :] = jnp.dot(
            vbuf_ref[t % 2].astype(jnp.bfloat16),
            w1c_ref[...],
            preferred_element_type=jnp.float32,
        ).astype(jnp.bfloat16)

    def _l1_dot(g):
        h = jnp.dot(
            adjb_ref[pl.ds(g * _G1, _G1), :],
            z1_ref[...],
            preferred_element_type=jnp.float32,
        )
        h = jnp.maximum(h, 0.0).astype(jnp.bfloat16)
        z2_ref[pl.ds(g * _G1, _G1), :] = jnp.dot(
            h, w2c_ref[...], preferred_element_type=jnp.float32
        ).astype(jnp.bfloat16)

    for t in range(nb):
        _wait_adj(t)
        adjb_ref[pl.ds(t * _RB, _RB), :] = abuf_ref[t % _NSLOT].astype(jnp.bfloat16)
        if t + _NSLOT - 1 < nb:
            _start_adj(t + _NSLOT - 1)
        if (t + 1) % (_G1 // _RB) == 0:
            g = (t + 1) // (_G1 // _RB) - 1
            if g >= 1:
                _l1_dot(g - 1)
    _l1_dot(ng1 - 1)

    out_ref[...] = bout_ref[...]
    for g in range(ng2):
        h = jnp.dot(
            adjb_ref[pl.ds(g * _G2, _G2), :],
            z2_ref[...],
            preferred_element_type=jnp.float32,
        )
        h = jnp.maximum(h, 0.0)
        x = jnp.sum(h, axis=1)
        contrib = jnp.sum(
            wout_ref[:, pl.ds(g * _G2, _G2)] * x[None, :], axis=1
        )
        out_ref[...] += contrib[None, :]


def kernel(v, adj, W1, W2, W_out, b_out):
    N, F_IN = v.shape
    H1 = W1.shape[1]
    H2 = W2.shape[1]
    LABEL = W_out.shape[0]

    out2d = pl.pallas_call(
        _kern,
        in_specs=[
            pl.BlockSpec(memory_space=pltpu.MemorySpace.HBM),
            pl.BlockSpec(memory_space=pltpu.MemorySpace.HBM),
            pl.BlockSpec(memory_space=pltpu.MemorySpace.VMEM),
            pl.BlockSpec(memory_space=pltpu.MemorySpace.VMEM),
            pl.BlockSpec(memory_space=pltpu.MemorySpace.VMEM),
            pl.BlockSpec(memory_space=pltpu.MemorySpace.VMEM),
        ],
        out_specs=pl.BlockSpec(memory_space=pltpu.MemorySpace.VMEM),
        out_shape=jax.ShapeDtypeStruct((1, LABEL), jnp.float32),
        scratch_shapes=[
            pltpu.VMEM((N, N), jnp.bfloat16),
            pltpu.VMEM((N, H1), jnp.bfloat16),
            pltpu.VMEM((N, H2), jnp.bfloat16),
            pltpu.VMEM((2, _VB, F_IN), jnp.float32),
            pltpu.VMEM((_NSLOT, _RB, N), jnp.float32),
            pltpu.VMEM((F_IN, H1), jnp.bfloat16),
            pltpu.VMEM((H1, H2), jnp.bfloat16),
            pltpu.SemaphoreType.DMA((2,)),
            pltpu.SemaphoreType.DMA((_NSLOT,)),
        ],
        compiler_params=pltpu.CompilerParams(
            vmem_limit_bytes=128 * 1024 * 1024,
        ),
    )(v, adj, W1, W2, W_out, b_out.reshape(1, LABEL))

    return out2d.reshape(LABEL)


# speedup vs baseline: 1.4105x; 1.4105x over previous
import jax
import jax.numpy as jnp
from jax.experimental import pallas as pl
from jax.experimental.pallas import tpu as pltpu


_RB = 256
_VB = 512
_SB = 512


def _kern(
    v_hbm,
    adj_hbm,
    w1_ref,
    w2_ref,
    wout_ref,
    bout_ref,
    out_ref,
    adjb_ref,
    z1_ref,
    z2_ref,
    vbuf_ref,
    abuf_ref,
    w1c_ref,
    w2c_ref,
    vsem,
    asem,
):
    N = adj_hbm.shape[0]
    nv = N // _VB
    nb = N // _RB
    ns = N // _SB

    pltpu.make_async_copy(
        adj_hbm.at[pl.ds(0, _RB), :], abuf_ref.at[0], asem.at[0]
    ).start()
    pltpu.make_async_copy(
        v_hbm.at[pl.ds(0, _VB), :], vbuf_ref.at[0], vsem.at[0]
    ).start()
    w1c_ref[...] = w1_ref[...].astype(jnp.bfloat16)
    w2c_ref[...] = w2_ref[...].astype(jnp.bfloat16)

    for t in range(nv):
        if t + 1 < nv:
            pltpu.make_async_copy(
                v_hbm.at[pl.ds((t + 1) * _VB, _VB), :],
                vbuf_ref.at[(t + 1) % 2],
                vsem.at[(t + 1) % 2],
            ).start()
        pltpu.make_async_copy(
            v_hbm.at[pl.ds(t * _VB, _VB), :], vbuf_ref.at[t % 2], vsem.at[t % 2]
        ).wait()
        z1_ref[pl.ds(t * _VB, _VB), :] = jnp.dot(
            vbuf_ref[t % 2].astype(jnp.bfloat16),
            w1c_ref[...],
            preferred_element_type=jnp.float32,
        ).astype(jnp.bfloat16)

    pltpu.make_async_copy(
        adj_hbm.at[pl.ds(_RB, _RB), :], abuf_ref.at[1], asem.at[1]
    ).start()

    def _l1_dot(b):
        h = jnp.dot(
            adjb_ref[pl.ds(b * _RB, _RB), :],
            z1_ref[...],
            preferred_element_type=jnp.float32,
        )
        h = jnp.maximum(h, 0.0).astype(jnp.bfloat16)
        z2_ref[pl.ds(b * _RB, _RB), :] = jnp.dot(
            h, w2c_ref[...], preferred_element_type=jnp.float32
        ).astype(jnp.bfloat16)

    for t in range(nb):
        pltpu.make_async_copy(
            adj_hbm.at[pl.ds(t * _RB, _RB), :], abuf_ref.at[t % 2], asem.at[t % 2]
        ).wait()
        adjb_ref[pl.ds(t * _RB, _RB), :] = abuf_ref[t % 2].astype(jnp.bfloat16)
        if t + 2 < nb:
            pltpu.make_async_copy(
                adj_hbm.at[pl.ds((t + 2) * _RB, _RB), :],
                abuf_ref.at[t % 2],
                asem.at[t % 2],
            ).start()
        if t >= 1:
            _l1_dot(t - 1)
    _l1_dot(nb - 1)

    out_ref[...] = bout_ref[...]
    for t in range(ns):
        h = jnp.dot(
            adjb_ref[pl.ds(t * _SB, _SB), :],
            z2_ref[...],
            preferred_element_type=jnp.float32,
        )
        h = jnp.maximum(h, 0.0)
        x = jnp.sum(h, axis=1)
        contrib = jnp.sum(
            wout_ref[:, pl.ds(t * _SB, _SB)] * x[None, :], axis=1
        )
        out_ref[...] += contrib[None, :]


def kernel(v, adj, W1, W2, W_out, b_out):
    N, F_IN = v.shape
    H1 = W1.shape[1]
    H2 = W2.shape[1]
    LABEL = W_out.shape[0]

    out2d = pl.pallas_call(
        _kern,
        in_specs=[
            pl.BlockSpec(memory_space=pltpu.MemorySpace.HBM),
            pl.BlockSpec(memory_space=pltpu.MemorySpace.HBM),
            pl.BlockSpec(memory_space=pltpu.MemorySpace.VMEM),
            pl.BlockSpec(memory_space=pltpu.MemorySpace.VMEM),
            pl.BlockSpec(memory_space=pltpu.MemorySpace.VMEM),
            pl.BlockSpec(memory_space=pltpu.MemorySpace.VMEM),
        ],
        out_specs=pl.BlockSpec(memory_space=pltpu.MemorySpace.VMEM),
        out_shape=jax.ShapeDtypeStruct((1, LABEL), jnp.float32),
        scratch_shapes=[
            pltpu.VMEM((N, N), jnp.bfloat16),
            pltpu.VMEM((N, H1), jnp.bfloat16),
            pltpu.VMEM((N, H2), jnp.bfloat16),
            pltpu.VMEM((2, _VB, F_IN), jnp.float32),
            pltpu.VMEM((2, _RB, N), jnp.float32),
            pltpu.VMEM((F_IN, H1), jnp.bfloat16),
            pltpu.VMEM((H1, H2), jnp.bfloat16),
            pltpu.SemaphoreType.DMA((2,)),
            pltpu.SemaphoreType.DMA((2,)),
        ],
        compiler_params=pltpu.CompilerParams(
            vmem_limit_bytes=128 * 1024 * 1024,
        ),
    )(v, adj, W1, W2, W_out, b_out.reshape(1, LABEL))

    return out2d.reshape(LABEL)
